# baseline (speedup 1.0000x reference)
import jax
import jax.numpy as jnp
from jax import lax
from jax.experimental import pallas as pl
from jax.experimental.pallas import tpu as pltpu
from jax.experimental.pallas import tpu_sc as plsc

_D, _T, _L = 8, 288, 256
_N = _D * _T * _L
_OUT = 64
_K = 7
_PAIR = _K * _K
_PROW = 32
_NW = 32
_ROWS_W = (_D * _T) // _NW
_PAIRS_W = _ROWS_W // 2
_PER_W = _ROWS_W * _L
_GROUPS = _L // 16


_GATHER_DNUMS = lax.GatherDimensionNumbers(
    offset_dims=(), collapsed_slice_dims=(0,), start_index_map=(0,))


def _splat(vec, j):
    idx = jnp.full((16, 1), j, jnp.int32)
    return lax.gather(vec, idx, _GATHER_DNUMS, (1,),
                      mode=lax.GatherScatterMode.PROMISE_IN_BOUNDS)


def _compute_row(inpall_v, row, tbl_v, out_v, iota):
    def group_body(g, _):
        w = inpall_v[row, pl.ds(g * 16, 16)]
        x0 = w & 255
        x1 = (w >> 8) & 255
        x2 = (w >> 16) & 255
        x3 = w >> 24
        a01 = (x0 * 7 + x1) * _PROW
        a23 = (x2 * 7 + x3) * _PROW + _PAIR * _PROW
        base = g * 16
        pend = []
        for e in range(2 * _PROW):
            a = a01 + e if e < _PROW else a23 + (e - _PROW)
            pend.append((e, plsc.load_gather(tbl_v, [a])))
            if len(pend) > 3:
                q, h = pend.pop(0)
                out_v[q, pl.ds(base, 16)] = h
        for q, h in pend:
            out_v[q, pl.ds(base, 16)] = h
        return 0

    lax.fori_loop(0, _GROUPS, group_body, 0)


def _sc_body(tbl_hbm, inp_hbm, out_hbm, tbl_v, inpall_v, out_v0, out_v1,
             sem_a, sem_b):
    wid = lax.axis_index("s") * 2 + lax.axis_index("c")
    d = wid // 4
    t0 = (wid % 4) * _ROWS_W
    iota = lax.broadcasted_iota(jnp.int32, (16,), 0)

    pltpu.sync_copy(tbl_hbm, tbl_v)
    pltpu.sync_copy(inp_hbm.at[d, pl.ds(t0, _ROWS_W), :], inpall_v)

    def pair_body(i, _):
        ta = t0 + 2 * i
        tb = ta + 1

        @pl.when(i > 0)
        def _():
            pltpu.make_async_copy(out_v0, out_hbm.at[d, ta], sem_a).wait()
            pltpu.make_async_copy(out_v1, out_hbm.at[d, tb], sem_b).wait()

        _compute_row(inpall_v, 2 * i, tbl_v, out_v0, iota)
        pltpu.async_copy(out_v0, out_hbm.at[d, ta], sem_a)
        _compute_row(inpall_v, 2 * i + 1, tbl_v, out_v1, iota)
        pltpu.async_copy(out_v1, out_hbm.at[d, tb], sem_b)
        return 0

    lax.fori_loop(0, _PAIRS_W, pair_body, 0)
    pltpu.make_async_copy(out_v0, out_hbm.at[d, t0], sem_a).wait()
    pltpu.make_async_copy(out_v1, out_hbm.at[d, t0 + 1], sem_b).wait()


@jax.jit
def _sc_lookup(tbl, inp):
    mesh = plsc.VectorSubcoreMesh(core_axis_name="c", subcore_axis_name="s")
    f = pl.kernel(
        _sc_body,
        mesh=mesh,
        out_type=jax.ShapeDtypeStruct((_D, _T, _OUT, _L), jnp.float32),
        scratch_types=[
            pltpu.VMEM((2 * _PAIR * _PROW,), jnp.float32),
            pltpu.VMEM((_ROWS_W, _L), jnp.int32),
            pltpu.VMEM((_OUT, _L), jnp.float32),
            pltpu.VMEM((_OUT, _L), jnp.float32),
            pltpu.SemaphoreType.DMA,
            pltpu.SemaphoreType.DMA,
        ],
        compiler_params=pltpu.CompilerParams(
            needs_layout_passes=False, use_tc_tiling_on_sc=True
        ),
    )
    return f(tbl, inp)


def kernel(inp, W_flow, W_day, W_time, W_loc):
    shape3 = (_K, _K, 16)
    p01 = jnp.concatenate(
        [
            jnp.broadcast_to(W_flow[:_K][:, None, :], shape3),
            jnp.broadcast_to(W_day[:_K][None, :, :], shape3),
        ],
        axis=-1,
    ).reshape(_PAIR * _PROW)
    p23 = jnp.concatenate(
        [
            jnp.broadcast_to(W_time[:_K][:, None, :], shape3),
            jnp.broadcast_to(W_loc[:_K][None, :, :], shape3),
        ],
        axis=-1,
    ).reshape(_PAIR * _PROW)
    tbl = jnp.concatenate([p01, p23])
    inp_packed = lax.bitcast_convert_type(inp.astype(jnp.int8), jnp.int32)
    return jnp.swapaxes(_sc_lookup(tbl, inp_packed), 2, 3)


# speedup vs baseline: 3.6473x; 3.6473x over previous
import jax
import jax.numpy as jnp
from jax import lax
from jax.experimental import pallas as pl
from jax.experimental.pallas import tpu as pltpu
from jax.experimental.pallas import tpu_sc as plsc

_D, _T, _L = 8, 288, 256
_N = _D * _T * _L
_OUT = 64
_K = 7
_PAIR = _K * _K
_PROW = 32
_PSTRIDE = 33
_NW = 32
_ROWS_W = (_D * _T) // _NW
_PAIRS_W = _ROWS_W // 2
_PER_W = _ROWS_W * _L
_GROUPS = _L // 16


_GATHER_DNUMS = lax.GatherDimensionNumbers(
    offset_dims=(), collapsed_slice_dims=(0,), start_index_map=(0,))


def _splat(vec, j):
    idx = jnp.full((16, 1), j, jnp.int32)
    return lax.gather(vec, idx, _GATHER_DNUMS, (1,),
                      mode=lax.GatherScatterMode.PROMISE_IN_BOUNDS)


def _compute_row(inpall_v, row, tbl_v, out_v, iota):
    def group_body(g, _):
        w = inpall_v[row, pl.ds(g * 16, 16)]
        x0 = w & 255
        x1 = (w >> 8) & 255
        x2 = (w >> 16) & 255
        x3 = w >> 24
        a01 = (x0 * 7 + x1) * _PSTRIDE
        a23 = (x2 * 7 + x3) * _PSTRIDE + _PAIR * _PSTRIDE
        base = g * 16
        pend = []
        for e in range(2 * _PROW):
            a = a01 + e if e < _PROW else a23 + (e - _PROW)
            pend.append((e, plsc.load_gather(tbl_v, [a])))
            if len(pend) > 3:
                q, h = pend.pop(0)
                out_v[q, pl.ds(base, 16)] = h
        for q, h in pend:
            out_v[q, pl.ds(base, 16)] = h
        return 0

    lax.fori_loop(0, _GROUPS, group_body, 0)


def _sc_body(tbl_hbm, inp_hbm, out_hbm, tbl_v, inpall_v, out_v0, out_v1,
             sem_a, sem_b):
    wid = lax.axis_index("s") * 2 + lax.axis_index("c")
    d = wid // 4
    t0 = (wid % 4) * _ROWS_W
    iota = lax.broadcasted_iota(jnp.int32, (16,), 0)

    pltpu.sync_copy(tbl_hbm, tbl_v)
    pltpu.sync_copy(inp_hbm.at[d, pl.ds(t0, _ROWS_W), :], inpall_v)

    def pair_body(i, _):
        ta = t0 + 2 * i
        tb = ta + 1

        @pl.when(i > 0)
        def _():
            pltpu.make_async_copy(out_v0, out_hbm.at[d, ta], sem_a).wait()
            pltpu.make_async_copy(out_v1, out_hbm.at[d, tb], sem_b).wait()

        _compute_row(inpall_v, 2 * i, tbl_v, out_v0, iota)
        pltpu.async_copy(out_v0, out_hbm.at[d, ta], sem_a)
        _compute_row(inpall_v, 2 * i + 1, tbl_v, out_v1, iota)
        pltpu.async_copy(out_v1, out_hbm.at[d, tb], sem_b)
        return 0

    lax.fori_loop(0, _PAIRS_W, pair_body, 0)
    pltpu.make_async_copy(out_v0, out_hbm.at[d, t0], sem_a).wait()
    pltpu.make_async_copy(out_v1, out_hbm.at[d, t0 + 1], sem_b).wait()


@jax.jit
def _sc_lookup(tbl, inp):
    mesh = plsc.VectorSubcoreMesh(core_axis_name="c", subcore_axis_name="s")
    f = pl.kernel(
        _sc_body,
        mesh=mesh,
        out_type=jax.ShapeDtypeStruct((_D, _T, _OUT, _L), jnp.float32),
        scratch_types=[
            pltpu.VMEM((2 * _PAIR * _PSTRIDE,), jnp.float32),
            pltpu.VMEM((_ROWS_W, _L), jnp.int32),
            pltpu.VMEM((_OUT, _L), jnp.float32),
            pltpu.VMEM((_OUT, _L), jnp.float32),
            pltpu.SemaphoreType.DMA,
            pltpu.SemaphoreType.DMA,
        ],
        compiler_params=pltpu.CompilerParams(
            needs_layout_passes=False, use_tc_tiling_on_sc=True
        ),
    )
    return f(tbl, inp)


def kernel(inp, W_flow, W_day, W_time, W_loc):
    shape3 = (_K, _K, 16)
    pad = ((0, 0), (0, _PSTRIDE - _PROW))
    p01 = jnp.pad(
        jnp.concatenate(
            [
                jnp.broadcast_to(W_flow[:_K][:, None, :], shape3),
                jnp.broadcast_to(W_day[:_K][None, :, :], shape3),
            ],
            axis=-1,
        ).reshape(_PAIR, _PROW),
        pad,
    ).reshape(_PAIR * _PSTRIDE)
    p23 = jnp.pad(
        jnp.concatenate(
            [
                jnp.broadcast_to(W_time[:_K][:, None, :], shape3),
                jnp.broadcast_to(W_loc[:_K][None, :, :], shape3),
            ],
            axis=-1,
        ).reshape(_PAIR, _PROW),
        pad,
    ).reshape(_PAIR * _PSTRIDE)
    tbl = jnp.concatenate([p01, p23])
    inp_packed = lax.bitcast_convert_type(inp.astype(jnp.int8), jnp.int32)
    return jnp.swapaxes(_sc_lookup(tbl, inp_packed), 2, 3)


# speedup vs baseline: 4.5073x; 1.2358x over previous
import jax
import jax.numpy as jnp
from jax import lax
from jax.experimental import pallas as pl
from jax.experimental.pallas import tpu as pltpu
from jax.experimental.pallas import tpu_sc as plsc

_D, _T, _L = 8, 288, 256
_N = _D * _T * _L
_OUT = 64
_K = 7
_PAIR = _K * _K
_PROW = 32
_PSTRIDE = 33
_NW = 32
_ROWS_W = (_D * _T) // _NW
_PAIRS_W = _ROWS_W // 2
_PER_W = _ROWS_W * _L
_GROUPS = _L // 16


_GATHER_DNUMS = lax.GatherDimensionNumbers(
    offset_dims=(), collapsed_slice_dims=(0,), start_index_map=(0,))


def _splat(vec, j):
    idx = jnp.full((16, 1), j, jnp.int32)
    return lax.gather(vec, idx, _GATHER_DNUMS, (1,),
                      mode=lax.GatherScatterMode.PROMISE_IN_BOUNDS)


def _compute_row(inpall_v, row, tbl_v, out_v, iota):
    def group_body(g, _):
        w = inpall_v[row, pl.ds(g * 16, 16)]
        x0 = w & 255
        x1 = (w >> 8) & 255
        x2 = (w >> 16) & 255
        x3 = w >> 24
        a01 = (x0 * 7 + x1) * _PSTRIDE
        a23 = (x2 * 7 + x3) * _PSTRIDE + _PAIR * _PSTRIDE
        base = g * 16
        pend = []
        for e in range(2 * _PROW):
            a = a01 + e if e < _PROW else a23 + (e - _PROW)
            pend.append((e, plsc.load_gather(tbl_v, [a])))
            if len(pend) > 6:
                q, h = pend.pop(0)
                out_v[q, pl.ds(base, 16)] = h
        for q, h in pend:
            out_v[q, pl.ds(base, 16)] = h
        return 0

    lax.fori_loop(0, _GROUPS, group_body, 0)


def _sc_body(tbl_hbm, inp_hbm, out_hbm, tbl_v, inpall_v, out_v0, out_v1,
             sem_a, sem_b):
    wid = lax.axis_index("s") * 2 + lax.axis_index("c")
    d = wid // 4
    t0 = (wid % 4) * _ROWS_W
    iota = lax.broadcasted_iota(jnp.int32, (16,), 0)

    pltpu.sync_copy(tbl_hbm, tbl_v)
    pltpu.sync_copy(inp_hbm.at[d, pl.ds(t0, _ROWS_W), :], inpall_v)

    def pair_body(i, _):
        ta = t0 + 2 * i
        tb = ta + 1

        @pl.when(i > 0)
        def _():
            pltpu.make_async_copy(out_v0, out_hbm.at[d, ta], sem_a).wait()
            pltpu.make_async_copy(out_v1, out_hbm.at[d, tb], sem_b).wait()

        _compute_row(inpall_v, 2 * i, tbl_v, out_v0, iota)
        pltpu.async_copy(out_v0, out_hbm.at[d, ta], sem_a)
        _compute_row(inpall_v, 2 * i + 1, tbl_v, out_v1, iota)
        pltpu.async_copy(out_v1, out_hbm.at[d, tb], sem_b)
        return 0

    lax.fori_loop(0, _PAIRS_W, pair_body, 0)
    pltpu.make_async_copy(out_v0, out_hbm.at[d, t0], sem_a).wait()
    pltpu.make_async_copy(out_v1, out_hbm.at[d, t0 + 1], sem_b).wait()


@jax.jit
def _sc_lookup(tbl, inp):
    mesh = plsc.VectorSubcoreMesh(core_axis_name="c", subcore_axis_name="s")
    f = pl.kernel(
        _sc_body,
        mesh=mesh,
        out_type=jax.ShapeDtypeStruct((_D, _T, _OUT, _L), jnp.float32),
        scratch_types=[
            pltpu.VMEM((2 * _PAIR * _PSTRIDE,), jnp.float32),
            pltpu.VMEM((_ROWS_W, _L), jnp.int32),
            pltpu.VMEM((_OUT, _L), jnp.float32),
            pltpu.VMEM((_OUT, _L), jnp.float32),
            pltpu.SemaphoreType.DMA,
            pltpu.SemaphoreType.DMA,
        ],
        compiler_params=pltpu.CompilerParams(
            needs_layout_passes=False, use_tc_tiling_on_sc=True
        ),
    )
    return f(tbl, inp)


def kernel(inp, W_flow, W_day, W_time, W_loc):
    shape3 = (_K, _K, 16)
    pad = ((0, 0), (0, _PSTRIDE - _PROW))
    p01 = jnp.pad(
        jnp.concatenate(
            [
                jnp.broadcast_to(W_flow[:_K][:, None, :], shape3),
                jnp.broadcast_to(W_day[:_K][None, :, :], shape3),
            ],
            axis=-1,
        ).reshape(_PAIR, _PROW),
        pad,
    ).reshape(_PAIR * _PSTRIDE)
    p23 = jnp.pad(
        jnp.concatenate(
            [
                jnp.broadcast_to(W_time[:_K][:, None, :], shape3),
                jnp.broadcast_to(W_loc[:_K][None, :, :], shape3),
            ],
            axis=-1,
        ).reshape(_PAIR, _PROW),
        pad,
    ).reshape(_PAIR * _PSTRIDE)
    tbl = jnp.concatenate([p01, p23])
    inp_packed = lax.bitcast_convert_type(inp.astype(jnp.int8), jnp.int32)
    return jnp.swapaxes(_sc_lookup(tbl, inp_packed), 2, 3)


# speedup vs baseline: 4.5293x; 1.0049x over previous
import jax
import jax.numpy as jnp
from jax import lax
from jax.experimental import pallas as pl
from jax.experimental.pallas import tpu as pltpu
from jax.experimental.pallas import tpu_sc as plsc

_D, _T, _L = 8, 288, 256
_N = _D * _T * _L
_OUT = 64
_K = 7
_PAIR = _K * _K
_PROW = 32
_PSTRIDE = 33
_NW = 32
_ROWS_W = (_D * _T) // _NW
_PAIRS_W = _ROWS_W // 2
_PER_W = _ROWS_W * _L
_GROUPS = _L // 16


_GATHER_DNUMS = lax.GatherDimensionNumbers(
    offset_dims=(), collapsed_slice_dims=(0,), start_index_map=(0,))


def _splat(vec, j):
    idx = jnp.full((16, 1), j, jnp.int32)
    return lax.gather(vec, idx, _GATHER_DNUMS, (1,),
                      mode=lax.GatherScatterMode.PROMISE_IN_BOUNDS)


def _compute_row(inpall_v, row, tbl_v, out_v, iota):
    def group_body(g, _):
        w = inpall_v[row, pl.ds(g * 16, 16)]
        x0 = w & 255
        x1 = (w >> 8) & 255
        x2 = (w >> 16) & 255
        x3 = w >> 24
        a01 = (x0 * 7 + x1) * _PSTRIDE
        a23 = (x2 * 7 + x3) * _PSTRIDE + _PAIR * _PSTRIDE
        base = g * 16
        pend = []
        for e in range(2 * _PROW):
            a = a01 + e if e < _PROW else a23 + (e - _PROW)
            pend.append((e, plsc.load_gather(tbl_v, [a])))
            if len(pend) > 10:
                q, h = pend.pop(0)
                out_v[q, pl.ds(base, 16)] = h
        for q, h in pend:
            out_v[q, pl.ds(base, 16)] = h
        return 0

    lax.fori_loop(0, _GROUPS, group_body, 0)


def _sc_body(tbl_hbm, inp_hbm, out_hbm, tbl_v, inpall_v, out_v0, out_v1,
             sem_a, sem_b):
    wid = lax.axis_index("s") * 2 + lax.axis_index("c")
    d = wid // 4
    t0 = (wid % 4) * _ROWS_W
    iota = lax.broadcasted_iota(jnp.int32, (16,), 0)

    pltpu.sync_copy(tbl_hbm, tbl_v)
    pltpu.sync_copy(inp_hbm.at[d, pl.ds(t0, _ROWS_W), :], inpall_v)

    def pair_body(i, _):
        ta = t0 + 2 * i
        tb = ta + 1

        @pl.when(i > 0)
        def _():
            pltpu.make_async_copy(out_v0, out_hbm.at[d, ta], sem_a).wait()
            pltpu.make_async_copy(out_v1, out_hbm.at[d, tb], sem_b).wait()

        _compute_row(inpall_v, 2 * i, tbl_v, out_v0, iota)
        pltpu.async_copy(out_v0, out_hbm.at[d, ta], sem_a)
        _compute_row(inpall_v, 2 * i + 1, tbl_v, out_v1, iota)
        pltpu.async_copy(out_v1, out_hbm.at[d, tb], sem_b)
        return 0

    lax.fori_loop(0, _PAIRS_W, pair_body, 0)
    pltpu.make_async_copy(out_v0, out_hbm.at[d, t0], sem_a).wait()
    pltpu.make_async_copy(out_v1, out_hbm.at[d, t0 + 1], sem_b).wait()


@jax.jit
def _sc_lookup(tbl, inp):
    mesh = plsc.VectorSubcoreMesh(core_axis_name="c", subcore_axis_name="s")
    f = pl.kernel(
        _sc_body,
        mesh=mesh,
        out_type=jax.ShapeDtypeStruct((_D, _T, _OUT, _L), jnp.float32),
        scratch_types=[
            pltpu.VMEM((2 * _PAIR * _PSTRIDE,), jnp.float32),
            pltpu.VMEM((_ROWS_W, _L), jnp.int32),
            pltpu.VMEM((_OUT, _L), jnp.float32),
            pltpu.VMEM((_OUT, _L), jnp.float32),
            pltpu.SemaphoreType.DMA,
            pltpu.SemaphoreType.DMA,
        ],
        compiler_params=pltpu.CompilerParams(
            needs_layout_passes=False, use_tc_tiling_on_sc=True
        ),
    )
    return f(tbl, inp)


def kernel(inp, W_flow, W_day, W_time, W_loc):
    shape3 = (_K, _K, 16)
    pad = ((0, 0), (0, _PSTRIDE - _PROW))
    p01 = jnp.pad(
        jnp.concatenate(
            [
                jnp.broadcast_to(W_flow[:_K][:, None, :], shape3),
                jnp.broadcast_to(W_day[:_K][None, :, :], shape3),
            ],
            axis=-1,
        ).reshape(_PAIR, _PROW),
        pad,
    ).reshape(_PAIR * _PSTRIDE)
    p23 = jnp.pad(
        jnp.concatenate(
            [
                jnp.broadcast_to(W_time[:_K][:, None, :], shape3),
                jnp.broadcast_to(W_loc[:_K][None, :, :], shape3),
            ],
            axis=-1,
        ).reshape(_PAIR, _PROW),
        pad,
    ).reshape(_PAIR * _PSTRIDE)
    tbl = jnp.concatenate([p01, p23])
    inp_packed = lax.bitcast_convert_type(inp.astype(jnp.int8), jnp.int32)
    return jnp.swapaxes(_sc_lookup(tbl, inp_packed), 2, 3)


# speedup vs baseline: 4.5377x; 1.0019x over previous
import jax
import jax.numpy as jnp
from jax import lax
from jax.experimental import pallas as pl
from jax.experimental.pallas import tpu as pltpu
from jax.experimental.pallas import tpu_sc as plsc

_D, _T, _L = 8, 288, 256
_N = _D * _T * _L
_OUT = 64
_K = 7
_PAIR = _K * _K
_PROW = 32
_PSTRIDE = 33
_NW = 32
_ROWS_W = (_D * _T) // _NW
_PAIRS_W = _ROWS_W // 2
_PER_W = _ROWS_W * _L
_GROUPS = _L // 16


def _compute_row(inpall_v, row, tbl_v, out_v):
    def group_body(g, _):
        w = inpall_v[row, pl.ds(g * 16, 16)]
        x0 = w & 255
        x1 = (w >> 8) & 255
        x2 = (w >> 16) & 255
        x3 = w >> 24
        a01 = (x0 * 7 + x1) * _PSTRIDE
        a23 = (x2 * 7 + x3) * _PSTRIDE + _PAIR * _PSTRIDE
        base = g * 16
        pend = []
        for e in range(2 * _PROW):
            a = a01 + e if e < _PROW else a23 + (e - _PROW)
            pend.append((e, plsc.load_gather(tbl_v, [a])))
            if len(pend) > 10:
                q, h = pend.pop(0)
                out_v[q, pl.ds(base, 16)] = h
        for q, h in pend:
            out_v[q, pl.ds(base, 16)] = h
        return 0

    lax.fori_loop(0, _GROUPS, group_body, 0)


def _sc_body(tbl_hbm, inp_hbm, out_hbm, tbl_v, inpall_v, out_v0, out_v1,
             sem_a, sem_b):
    wid = lax.axis_index("s") * 2 + lax.axis_index("c")
    d = wid // 4
    t0 = (wid % 4) * _ROWS_W

    pltpu.sync_copy(tbl_hbm, tbl_v)
    pltpu.sync_copy(inp_hbm.at[d, pl.ds(t0, _ROWS_W), :], inpall_v)

    def pair_body(i, _):
        ta = t0 + 2 * i
        tb = ta + 1

        @pl.when(i > 0)
        def _():
            pltpu.make_async_copy(out_v0, out_hbm.at[d, ta], sem_a).wait()
            pltpu.make_async_copy(out_v1, out_hbm.at[d, tb], sem_b).wait()

        _compute_row(inpall_v, 2 * i, tbl_v, out_v0)
        pltpu.async_copy(out_v0, out_hbm.at[d, ta], sem_a)
        _compute_row(inpall_v, 2 * i + 1, tbl_v, out_v1)
        pltpu.async_copy(out_v1, out_hbm.at[d, tb], sem_b)
        return 0

    lax.fori_loop(0, _PAIRS_W, pair_body, 0)
    pltpu.make_async_copy(out_v0, out_hbm.at[d, t0], sem_a).wait()
    pltpu.make_async_copy(out_v1, out_hbm.at[d, t0 + 1], sem_b).wait()


@jax.jit
def _sc_lookup(tbl, inp):
    mesh = plsc.VectorSubcoreMesh(core_axis_name="c", subcore_axis_name="s")
    f = pl.kernel(
        _sc_body,
        mesh=mesh,
        out_type=jax.ShapeDtypeStruct((_D, _T, _OUT, _L), jnp.float32),
        scratch_types=[
            pltpu.VMEM((2 * _PAIR * _PSTRIDE,), jnp.float32),
            pltpu.VMEM((_ROWS_W, _L), jnp.int32),
            pltpu.VMEM((_OUT, _L), jnp.float32),
            pltpu.VMEM((_OUT, _L), jnp.float32),
            pltpu.SemaphoreType.DMA,
            pltpu.SemaphoreType.DMA,
        ],
        compiler_params=pltpu.CompilerParams(
            needs_layout_passes=False, use_tc_tiling_on_sc=True
        ),
    )
    return f(tbl, inp)


def kernel(inp, W_flow, W_day, W_time, W_loc):
    shape3 = (_K, _K, 16)
    pad = ((0, 0), (0, _PSTRIDE - _PROW))
    p01 = jnp.pad(
        jnp.concatenate(
            [
                jnp.broadcast_to(W_flow[:_K][:, None, :], shape3),
                jnp.broadcast_to(W_day[:_K][None, :, :], shape3),
            ],
            axis=-1,
        ).reshape(_PAIR, _PROW),
        pad,
    ).reshape(_PAIR * _PSTRIDE)
    p23 = jnp.pad(
        jnp.concatenate(
            [
                jnp.broadcast_to(W_time[:_K][:, None, :], shape3),
                jnp.broadcast_to(W_loc[:_K][None, :, :], shape3),
            ],
            axis=-1,
        ).reshape(_PAIR, _PROW),
        pad,
    ).reshape(_PAIR * _PSTRIDE)
    tbl = jnp.concatenate([p01, p23])
    inp_packed = lax.bitcast_convert_type(inp.astype(jnp.int8), jnp.int32)
    return jnp.swapaxes(_sc_lookup(tbl, inp_packed), 2, 3)
